# Initial kernel scaffold; baseline (speedup 1.0000x reference)
import jax
import jax.numpy as jnp
from jax.experimental import pallas as pl


def kernel(news_representation, category, subCategory, category_embedding, subCategory_embedding):
    raise NotImplementedError("write your pallas kernel here")


# speedup vs baseline: 1.0610x; 1.0610x over previous
import functools

import jax
import jax.numpy as jnp
from jax.experimental import pallas as pl
from jax.experimental.pallas import tpu as pltpu

B = 4096
L = 50
D_NEWS = 400
CAT_NUM = 18
SUBCAT_NUM = 285
CAT_DIM = 100
SUBCAT_DIM = 100
D_OUT = D_NEWS + CAT_DIM + SUBCAT_DIM

_R = 512


def _fused_body(cat_ref, sub_ref, news_ref, cat_tab_ref, sub_tab_ref, out_ref):
    news = news_ref[...]
    cat = cat_ref[0, 0, :]
    sub = sub_ref[0, 0, :]

    cat_oh = (jax.lax.broadcasted_iota(jnp.int32, (CAT_NUM, _R), 0)
              == cat[None, :]).astype(jnp.float32)
    sub_oh = (jax.lax.broadcasted_iota(jnp.int32, (SUBCAT_NUM, _R), 0)
              == sub[None, :]).astype(jnp.float32)
    dn = (((0,), (0,)), ((), ()))
    cat_rep = jax.lax.dot_general(cat_oh, cat_tab_ref[...], dn,
                                  preferred_element_type=jnp.float32)
    sub_rep = jax.lax.dot_general(sub_oh, sub_tab_ref[...], dn,
                                  preferred_element_type=jnp.float32)

    out_ref[:, 0:D_NEWS] = news
    out_ref[:, D_NEWS:D_NEWS + CAT_DIM] = cat_rep
    out_ref[:, D_NEWS + CAT_DIM:D_OUT] = sub_rep


@functools.partial(jax.jit, static_argnames=("interpret",))
def _fused_call(news2, cat3, sub3, cat_tab, sub_tab, interpret=False):
    n_blocks = (B * L) // _R
    return pl.pallas_call(
        _fused_body,
        grid=(n_blocks,),
        in_specs=[
            pl.BlockSpec((1, 1, _R), lambda i: (i, 0, 0)),
            pl.BlockSpec((1, 1, _R), lambda i: (i, 0, 0)),
            pl.BlockSpec((_R, D_NEWS), lambda i: (i, 0)),
            pl.BlockSpec((CAT_NUM, CAT_DIM), lambda i: (0, 0)),
            pl.BlockSpec((SUBCAT_NUM, SUBCAT_DIM), lambda i: (0, 0)),
        ],
        out_specs=pl.BlockSpec((_R, D_OUT), lambda i: (i, 0)),
        out_shape=jax.ShapeDtypeStruct((B * L, D_OUT), jnp.float32),
        interpret=interpret,
    )(cat3, sub3, news2, cat_tab, sub_tab)


def kernel(news_representation, category, subCategory, category_embedding,
           subCategory_embedding, *, interpret=False):
    n_blocks = (B * L) // _R
    news2 = news_representation.reshape(B * L, D_NEWS)
    cat3 = category.astype(jnp.int32).reshape(n_blocks, 1, _R)
    sub3 = subCategory.astype(jnp.int32).reshape(n_blocks, 1, _R)
    out2 = _fused_call(news2, cat3, sub3, category_embedding,
                       subCategory_embedding, interpret=interpret)
    return out2.reshape(B, L, D_OUT)


# speedup vs baseline: 1.4384x; 1.3557x over previous
import functools

import jax
import jax.numpy as jnp
from jax.experimental import pallas as pl
from jax.experimental.pallas import tpu as pltpu

B = 4096
L = 50
D_NEWS = 400
CAT_NUM = 18
SUBCAT_NUM = 285
CAT_DIM = 100
SUBCAT_DIM = 100
D_OUT = D_NEWS + CAT_DIM + SUBCAT_DIM

_RB = 8


def _fused_body(cat_ref, sub_ref, news_ref, cat_tab_ref, sub_tab_ref, out_ref):
    out_ref[:, :, 0:D_NEWS] = news_ref[...]
    dn = (((0,), (0,)), ((), ()))
    for i in range(_RB):
        cat_oh = (jax.lax.broadcasted_iota(jnp.int32, (CAT_NUM, L), 0)
                  == cat_ref[i]).astype(jnp.float32)
        sub_oh = (jax.lax.broadcasted_iota(jnp.int32, (SUBCAT_NUM, L), 0)
                  == sub_ref[i]).astype(jnp.float32)
        cat_rep = jax.lax.dot_general(cat_oh, cat_tab_ref[...], dn,
                                      preferred_element_type=jnp.float32)
        sub_rep = jax.lax.dot_general(sub_oh, sub_tab_ref[...], dn,
                                      preferred_element_type=jnp.float32)
        out_ref[i, :, D_NEWS:D_NEWS + CAT_DIM] = cat_rep
        out_ref[i, :, D_NEWS + CAT_DIM:D_OUT] = sub_rep


@functools.partial(jax.jit, static_argnames=("interpret",))
def _fused_call(news, cat3, sub3, cat_tab, sub_tab, interpret=False):
    return pl.pallas_call(
        _fused_body,
        grid=(B // _RB,),
        in_specs=[
            pl.BlockSpec((_RB, 1, L), lambda i: (i, 0, 0)),
            pl.BlockSpec((_RB, 1, L), lambda i: (i, 0, 0)),
            pl.BlockSpec((_RB, L, D_NEWS), lambda i: (i, 0, 0)),
            pl.BlockSpec((CAT_NUM, CAT_DIM), lambda i: (0, 0)),
            pl.BlockSpec((SUBCAT_NUM, SUBCAT_DIM), lambda i: (0, 0)),
        ],
        out_specs=pl.BlockSpec((_RB, L, D_OUT), lambda i: (i, 0, 0)),
        out_shape=jax.ShapeDtypeStruct((B, L, D_OUT), jnp.float32),
        interpret=interpret,
    )(cat3, sub3, news, cat_tab, sub_tab)


def kernel(news_representation, category, subCategory, category_embedding,
           subCategory_embedding, *, interpret=False):
    cat3 = category.astype(jnp.int32).reshape(B, 1, L)
    sub3 = subCategory.astype(jnp.int32).reshape(B, 1, L)
    return _fused_call(news_representation, cat3, sub3, category_embedding,
                       subCategory_embedding, interpret=interpret)


# speedup vs baseline: 1.6116x; 1.1204x over previous
import functools

import jax
import jax.numpy as jnp
from jax.experimental import pallas as pl
from jax.experimental.pallas import tpu as pltpu

B = 4096
L = 50
D_NEWS = 400
CAT_NUM = 18
SUBCAT_NUM = 285
CAT_DIM = 100
SUBCAT_DIM = 100
D_OUT = D_NEWS + CAT_DIM + SUBCAT_DIM

_RB = 16


def _fused_body(cat_ref, sub_ref, news_ref, cat_tab_ref, sub_tab_ref, out_ref):
    out_ref[:, :, 0:D_NEWS] = news_ref[...]
    dn = (((0,), (0,)), ((), ()))
    for i in range(_RB):
        cat_oh = (jax.lax.broadcasted_iota(jnp.int32, (CAT_NUM, L), 0)
                  == cat_ref[i]).astype(jnp.float32)
        sub_oh = (jax.lax.broadcasted_iota(jnp.int32, (SUBCAT_NUM, L), 0)
                  == sub_ref[i]).astype(jnp.float32)
        cat_rep = jax.lax.dot_general(cat_oh, cat_tab_ref[...], dn,
                                      preferred_element_type=jnp.float32)
        sub_rep = jax.lax.dot_general(sub_oh, sub_tab_ref[...], dn,
                                      preferred_element_type=jnp.float32)
        out_ref[i, :, D_NEWS:D_NEWS + CAT_DIM] = cat_rep
        out_ref[i, :, D_NEWS + CAT_DIM:D_OUT] = sub_rep


@functools.partial(jax.jit, static_argnames=("interpret",))
def _fused_call(news, cat3, sub3, cat_tab, sub_tab, interpret=False):
    return pl.pallas_call(
        _fused_body,
        grid=(B // _RB,),
        in_specs=[
            pl.BlockSpec((_RB, 1, L), lambda i: (i, 0, 0)),
            pl.BlockSpec((_RB, 1, L), lambda i: (i, 0, 0)),
            pl.BlockSpec((_RB, L, D_NEWS), lambda i: (i, 0, 0)),
            pl.BlockSpec((CAT_NUM, CAT_DIM), lambda i: (0, 0)),
            pl.BlockSpec((SUBCAT_NUM, SUBCAT_DIM), lambda i: (0, 0)),
        ],
        out_specs=pl.BlockSpec((_RB, L, D_OUT), lambda i: (i, 0, 0)),
        out_shape=jax.ShapeDtypeStruct((B, L, D_OUT), jnp.float32),
        interpret=interpret,
    )(cat3, sub3, news, cat_tab, sub_tab)


def kernel(news_representation, category, subCategory, category_embedding,
           subCategory_embedding, *, interpret=False):
    cat3 = category.astype(jnp.int32).reshape(B, 1, L)
    sub3 = subCategory.astype(jnp.int32).reshape(B, 1, L)
    return _fused_call(news_representation, cat3, sub3, category_embedding,
                       subCategory_embedding, interpret=interpret)


# speedup vs baseline: 1.6998x; 1.0547x over previous
import functools

import jax
import jax.numpy as jnp
from jax.experimental import pallas as pl
from jax.experimental.pallas import tpu as pltpu

B = 4096
L = 50
D_NEWS = 400
CAT_NUM = 18
SUBCAT_NUM = 285
CAT_DIM = 100
SUBCAT_DIM = 100
D_OUT = D_NEWS + CAT_DIM + SUBCAT_DIM

_RB = 32


def _fused_body(cat_ref, sub_ref, news_ref, cat_tab_ref, sub_tab_ref, out_ref):
    out_ref[:, :, 0:D_NEWS] = news_ref[...]
    dn = (((0,), (0,)), ((), ()))
    for i in range(_RB):
        cat_oh = (jax.lax.broadcasted_iota(jnp.int32, (CAT_NUM, L), 0)
                  == cat_ref[i]).astype(jnp.float32)
        sub_oh = (jax.lax.broadcasted_iota(jnp.int32, (SUBCAT_NUM, L), 0)
                  == sub_ref[i]).astype(jnp.float32)
        cat_rep = jax.lax.dot_general(cat_oh, cat_tab_ref[...], dn,
                                      preferred_element_type=jnp.float32)
        sub_rep = jax.lax.dot_general(sub_oh, sub_tab_ref[...], dn,
                                      preferred_element_type=jnp.float32)
        out_ref[i, :, D_NEWS:D_NEWS + CAT_DIM] = cat_rep
        out_ref[i, :, D_NEWS + CAT_DIM:D_OUT] = sub_rep


@functools.partial(jax.jit, static_argnames=("interpret",))
def _fused_call(news, cat3, sub3, cat_tab, sub_tab, interpret=False):
    return pl.pallas_call(
        _fused_body,
        grid=(B // _RB,),
        in_specs=[
            pl.BlockSpec((_RB, 1, L), lambda i: (i, 0, 0)),
            pl.BlockSpec((_RB, 1, L), lambda i: (i, 0, 0)),
            pl.BlockSpec((_RB, L, D_NEWS), lambda i: (i, 0, 0)),
            pl.BlockSpec((CAT_NUM, CAT_DIM), lambda i: (0, 0)),
            pl.BlockSpec((SUBCAT_NUM, SUBCAT_DIM), lambda i: (0, 0)),
        ],
        out_specs=pl.BlockSpec((_RB, L, D_OUT), lambda i: (i, 0, 0)),
        out_shape=jax.ShapeDtypeStruct((B, L, D_OUT), jnp.float32),
        interpret=interpret,
    )(cat3, sub3, news, cat_tab, sub_tab)


def kernel(news_representation, category, subCategory, category_embedding,
           subCategory_embedding, *, interpret=False):
    cat3 = category.astype(jnp.int32).reshape(B, 1, L)
    sub3 = subCategory.astype(jnp.int32).reshape(B, 1, L)
    return _fused_call(news_representation, cat3, sub3, category_embedding,
                       subCategory_embedding, interpret=interpret)


# speedup vs baseline: 1.7174x; 1.0103x over previous
import functools

import jax
import jax.numpy as jnp
from jax.experimental import pallas as pl
from jax.experimental.pallas import tpu as pltpu

B = 4096
L = 50
D_NEWS = 400
CAT_NUM = 18
SUBCAT_NUM = 285
CAT_DIM = 100
SUBCAT_DIM = 100
D_OUT = D_NEWS + CAT_DIM + SUBCAT_DIM

_RB = 64


def _fused_body(cat_ref, sub_ref, news_ref, cat_tab_ref, sub_tab_ref, out_ref):
    out_ref[:, :, 0:D_NEWS] = news_ref[...]
    dn = (((0,), (0,)), ((), ()))
    for i in range(_RB):
        cat_oh = (jax.lax.broadcasted_iota(jnp.int32, (CAT_NUM, L), 0)
                  == cat_ref[i]).astype(jnp.float32)
        sub_oh = (jax.lax.broadcasted_iota(jnp.int32, (SUBCAT_NUM, L), 0)
                  == sub_ref[i]).astype(jnp.float32)
        cat_rep = jax.lax.dot_general(cat_oh, cat_tab_ref[...], dn,
                                      preferred_element_type=jnp.float32)
        sub_rep = jax.lax.dot_general(sub_oh, sub_tab_ref[...], dn,
                                      preferred_element_type=jnp.float32)
        out_ref[i, :, D_NEWS:D_NEWS + CAT_DIM] = cat_rep
        out_ref[i, :, D_NEWS + CAT_DIM:D_OUT] = sub_rep


@functools.partial(jax.jit, static_argnames=("interpret",))
def _fused_call(news, cat3, sub3, cat_tab, sub_tab, interpret=False):
    return pl.pallas_call(
        _fused_body,
        grid=(B // _RB,),
        in_specs=[
            pl.BlockSpec((_RB, 1, L), lambda i: (i, 0, 0)),
            pl.BlockSpec((_RB, 1, L), lambda i: (i, 0, 0)),
            pl.BlockSpec((_RB, L, D_NEWS), lambda i: (i, 0, 0)),
            pl.BlockSpec((CAT_NUM, CAT_DIM), lambda i: (0, 0)),
            pl.BlockSpec((SUBCAT_NUM, SUBCAT_DIM), lambda i: (0, 0)),
        ],
        out_specs=pl.BlockSpec((_RB, L, D_OUT), lambda i: (i, 0, 0)),
        out_shape=jax.ShapeDtypeStruct((B, L, D_OUT), jnp.float32),
        interpret=interpret,
    )(cat3, sub3, news, cat_tab, sub_tab)


def kernel(news_representation, category, subCategory, category_embedding,
           subCategory_embedding, *, interpret=False):
    cat3 = category.astype(jnp.int32).reshape(B, 1, L)
    sub3 = subCategory.astype(jnp.int32).reshape(B, 1, L)
    return _fused_call(news_representation, cat3, sub3, category_embedding,
                       subCategory_embedding, interpret=interpret)
